# Initial kernel scaffold; baseline (speedup 1.0000x reference)
import jax
import jax.numpy as jnp
from jax.experimental import pallas as pl


def kernel(des, tweet, num_prop, cat_prop, new_feature, edge_index, Wd, bd, Wt, bt, Wn, bn, Wc, bc, Wf, bf, Wi, bi, W1, as1, ad1, bg1, W2, as2, ad2, bg2, Wo1, bo1, Wo2, bo2):
    raise NotImplementedError("write your pallas kernel here")


# speedup vs baseline: 14.5437x; 14.5437x over previous
import functools
import jax
import jax.numpy as jnp
from jax import lax
from jax.experimental import pallas as pl
from jax.experimental.pallas import tpu as pltpu
from jax.experimental.pallas import tpu_sc as plsc

NND = 10000
NPAD = 10240
NEDGE = 320000
EPAD = 330240
NW = 32
TPT = EPAD // NW
NCHUNK = TPT // 16
SUPER = 15
NSUP = NCHUNK // SUPER
ACCW = 128
STRIPE = NPAD // 16
BLK = 512
GRID = NPAD // BLK

_f32 = jnp.float32


def _lrelu(v, s=0.01):
    return jnp.maximum(v, v * s)


def _enc_body(des, tweet, num, cat, nf,
              Wd, bd, Wt, bt, Wn, bn, Wc, bc, Wf, bf, Wi, bi,
              W1, As1, Ad1,
              h0, h1o, h2o, h3o, alsT, aldT):
    d = _lrelu(jnp.dot(des[...], Wd[...], preferred_element_type=_f32) + bd[...])
    t = _lrelu(jnp.dot(tweet[...], Wt[...], preferred_element_type=_f32) + bt[...])
    n = _lrelu(jnp.dot(num[...], Wn[...], preferred_element_type=_f32) + bn[...])
    c = _lrelu(jnp.dot(cat[...], Wc[...], preferred_element_type=_f32) + bc[...])
    f = _lrelu(jnp.dot(nf[...], Wf[...], preferred_element_type=_f32) + bf[...])
    x = jnp.concatenate([d, t, n, c, f], axis=1)
    x = _lrelu(jnp.dot(x, Wi[...], preferred_element_type=_f32) + bi[...])
    h = jnp.dot(x, W1[...], preferred_element_type=_f32)
    h0[...] = h[:, 0:128]
    h1o[...] = h[:, 128:256]
    h2o[...] = h[:, 256:384]
    h3o[...] = h[:, 384:512]
    alsT[...] = lax.dot_general(As1[...], x, (((0,), (1,)), ((), ())),
                                preferred_element_type=_f32)
    aldT[...] = lax.dot_general(Ad1[...], x, (((0,), (1,)), ((), ())),
                                preferred_element_type=_f32)


def _full(shape):
    return pl.BlockSpec(shape, lambda i: (0,) * len(shape))


def _enc_call(des, tweet, num, cat, nf, Wd, bd, Wt, bt, Wn, bn, Wc, bc,
              Wf, bf, Wi, bi, W1, As1, Ad1):
    row = lambda w: pl.BlockSpec((BLK, w), lambda i: (i, 0))
    return pl.pallas_call(
        _enc_body,
        grid=(GRID,),
        in_specs=[row(768), row(768), row(7), row(11), row(1),
                  _full((768, 28)), _full((1, 28)),
                  _full((768, 36)), _full((1, 36)),
                  _full((7, 12)), _full((1, 12)),
                  _full((11, 40)), _full((1, 40)),
                  _full((1, 12)), _full((1, 12)),
                  _full((128, 128)), _full((1, 128)),
                  _full((128, 512)), _full((128, 4)), _full((128, 4))],
        out_specs=[row(128), row(128), row(128), row(128),
                   pl.BlockSpec((4, BLK), lambda i: (0, i)),
                   pl.BlockSpec((4, BLK), lambda i: (0, i))],
        out_shape=[jax.ShapeDtypeStruct((NPAD, 128), _f32)] * 4 +
                  [jax.ShapeDtypeStruct((4, NPAD), _f32)] * 2,
    )(des, tweet, num, cat, nf, Wd, bd, Wt, bt, Wn, bn, Wc, bc, Wf, bf,
      Wi, bi, W1, As1, Ad1)


def _make_gat_sc(heads):
    mesh = plsc.VectorSubcoreMesh(core_axis_name="c", subcore_axis_name="s")
    scratch = [
        pltpu.VMEM((NPAD,), _f32),
        pltpu.VMEM((NPAD,), _f32),
        pltpu.VMEM((SUPER * 16,), jnp.int32),
        pltpu.VMEM((SUPER * 16,), jnp.int32),
        pltpu.VMEM((16, 128), _f32),
        pltpu.VMEM((16, 128), _f32),
        pltpu.VMEM((NPAD,), _f32),
        pltpu.VMEM_SHARED((NPAD, ACCW), _f32),
        pltpu.SemaphoreType.DMA,
    ]

    def body(src_hbm, dst_hbm, alsT_hbm, aldT_hbm, *rest):
        tables = rest[:heads]
        zr_hbm = rest[heads]
        zd_hbm = rest[heads + 1]
        out_hbm = rest[heads + 2]
        den_hbm = rest[heads + 3]
        (als_v, ald_v, src_v, dst_v, rows_g, sbuf, den_v, acc_sh,
         sem) = rest[heads + 4:]
        cid = lax.axis_index("c")
        sid = lax.axis_index("s")
        w = cid * 16 + sid
        ebase = w * TPT

        for h in range(heads):
            pltpu.sync_copy(alsT_hbm.at[h], als_v)
            pltpu.sync_copy(aldT_hbm.at[h], ald_v)
            pltpu.sync_copy(zr_hbm, acc_sh.at[pl.ds(sid * STRIPE, STRIPE)])
            pltpu.sync_copy(zd_hbm, den_v)
            plsc.subcore_barrier()
            tab = tables[h]

            def superchunk(si, carry):
                pltpu.sync_copy(
                    src_hbm.at[pl.ds(ebase + si * (SUPER * 16), SUPER * 16)],
                    src_v)
                pltpu.sync_copy(
                    dst_hbm.at[pl.ds(ebase + si * (SUPER * 16), SUPER * 16)],
                    dst_v)

                def chunk(cc, carry2):
                    off = cc * 16
                    sidx = src_v[pl.ds(off, 16)]
                    didx = dst_v[pl.ds(off, 16)]
                    a = (plsc.load_gather(als_v, [sidx]) +
                         plsc.load_gather(ald_v, [didx]))
                    a = jnp.maximum(a, a * 0.2)
                    ex = jnp.exp(a)
                    pltpu.async_copy(tab.at[sidx], rows_g, sem).wait()
                    lanes = lax.iota(jnp.int32, 16)
                    for e in range(16):
                        s = ex[e]
                        for j in range(8):
                            sl = pl.ds(j * 16, 16)
                            sbuf[e, sl] = rows_g[e, sl] * s
                        plsc.addupdate_scatter(den_v, [didx], ex,
                                               mask=lanes == e)
                    pltpu.sync_copy(sbuf, acc_sh.at[didx], add=True)
                    return carry2

                lax.fori_loop(0, SUPER, chunk, 0)
                return carry

            lax.fori_loop(0, NSUP, superchunk, 0)
            plsc.subcore_barrier()
            pltpu.sync_copy(acc_sh.at[pl.ds(sid * STRIPE, STRIPE)],
                            out_hbm.at[h, cid, pl.ds(sid * STRIPE, STRIPE)])
            pltpu.sync_copy(den_v, den_hbm.at[h, w])
            plsc.subcore_barrier()

    return pl.kernel(
        body,
        out_type=(jax.ShapeDtypeStruct((heads, 2, NPAD, ACCW), _f32),
                  jax.ShapeDtypeStruct((heads, NW, NPAD), _f32)),
        mesh=mesh,
        scratch_types=scratch,
        compiler_params=pltpu.CompilerParams(needs_layout_passes=False),
    )


def _mid_body(acc, dnn, bg1, W2, As2, Ad2, h2o, alsT, aldT):
    a = acc[...]
    d = dnn[...]
    xs = []
    for h in range(4):
        num = a[h, 0] + a[h, 1]
        den = jnp.sum(d[h], axis=0).reshape(BLK, 1)
        xs.append(num / (den + 1e-16))
    x1 = jnp.concatenate(xs, axis=1) + bg1[...]
    h2o[...] = jnp.dot(x1, W2[...], preferred_element_type=_f32)
    alsT[...] = lax.dot_general(As2[...], x1, (((0,), (1,)), ((), ())),
                                preferred_element_type=_f32)
    aldT[...] = lax.dot_general(Ad2[...], x1, (((0,), (1,)), ((), ())),
                                preferred_element_type=_f32)


def _mid_call(acc, dnn, bg1, W2, As2, Ad2):
    return pl.pallas_call(
        _mid_body,
        grid=(GRID,),
        in_specs=[pl.BlockSpec((4, 2, BLK, ACCW), lambda i: (0, 0, i, 0)),
                  pl.BlockSpec((4, NW, BLK), lambda i: (0, 0, i)),
                  _full((1, 512)), _full((512, 128)),
                  _full((512, 1)), _full((512, 1))],
        out_specs=[pl.BlockSpec((BLK, 128), lambda i: (i, 0)),
                   pl.BlockSpec((1, BLK), lambda i: (0, i)),
                   pl.BlockSpec((1, BLK), lambda i: (0, i))],
        out_shape=[jax.ShapeDtypeStruct((NPAD, 128), _f32),
                   jax.ShapeDtypeStruct((1, NPAD), _f32),
                   jax.ShapeDtypeStruct((1, NPAD), _f32)],
    )(acc, dnn, bg1, W2, As2, Ad2)


def _out_body(acc, dnn, bg2, Wo1, bo1, Wo2, bo2, y):
    a = acc[...]
    num = a[0, 0] + a[0, 1]
    den = jnp.sum(dnn[...][0], axis=0).reshape(BLK, 1)
    x2 = num / (den + 1e-16) + bg2[...]
    v = _lrelu(jnp.dot(x2, Wo1[...], preferred_element_type=_f32) + bo1[...])
    y[...] = jnp.dot(v, Wo2[...], preferred_element_type=_f32) + bo2[...]


def _out_call(acc, dnn, bg2, Wo1, bo1, Wo2, bo2):
    return pl.pallas_call(
        _out_body,
        grid=(GRID,),
        in_specs=[pl.BlockSpec((1, 2, BLK, ACCW), lambda i: (0, 0, i, 0)),
                  pl.BlockSpec((1, NW, BLK), lambda i: (0, 0, i)),
                  _full((1, 128)), _full((128, 128)), _full((1, 128)),
                  _full((128, 2)), _full((1, 2))],
        out_specs=pl.BlockSpec((BLK, 2), lambda i: (i, 0)),
        out_shape=jax.ShapeDtypeStruct((NPAD, 2), _f32),
    )(acc, dnn, bg2, Wo1, bo1, Wo2, bo2)


@jax.jit
def kernel(des, tweet, num_prop, cat_prop, new_feature, edge_index,
           Wd, bd, Wt, bt, Wn, bn, Wc, bc, Wf, bf, Wi, bi,
           W1, as1, ad1, bg1, W2, as2, ad2, bg2, Wo1, bo1, Wo2, bo2):
    f32 = _f32
    padn = lambda v: jnp.pad(v, ((0, NPAD - NND), (0, 0)))
    des_p = padn(des)
    tweet_p = padn(tweet)
    num_p = padn(num_prop)
    cat_p = padn(cat_prop)
    nf_p = padn(new_feature)

    loop = jnp.arange(NND, dtype=jnp.int32)
    src = jnp.concatenate([edge_index[0].astype(jnp.int32), loop,
                           jnp.zeros((EPAD - NEDGE - NND,), jnp.int32)])
    dst = jnp.concatenate([edge_index[1].astype(jnp.int32), loop,
                           jnp.full((EPAD - NEDGE - NND,), NND, jnp.int32)])

    As1 = (W1.reshape(128, 4, 128) * as1).sum(-1)
    Ad1 = (W1.reshape(128, 4, 128) * ad1).sum(-1)
    As2 = (W2.reshape(512, 1, 128) * as2).sum(-1)
    Ad2 = (W2.reshape(512, 1, 128) * ad2).sum(-1)

    r1 = lambda b: b.reshape(1, -1).astype(f32)
    h0, h1, h2, h3, alsT1, aldT1 = _enc_call(
        des_p, tweet_p, num_p, cat_p, nf_p,
        Wd, r1(bd), Wt, r1(bt), Wn, r1(bn), Wc, r1(bc), Wf, r1(bf),
        Wi, r1(bi), W1, As1, Ad1)

    zr = jnp.zeros((STRIPE, ACCW), f32)
    zd = jnp.zeros((NPAD,), f32)
    acc1, den1 = _make_gat_sc(4)(src, dst, alsT1, aldT1, h0, h1, h2, h3,
                                 zr, zd)

    hx, alsT2, aldT2 = _mid_call(acc1, den1, r1(bg1), W2, As2, Ad2)

    acc2, den2 = _make_gat_sc(1)(src, dst, alsT2, aldT2, hx, zr, zd)

    y = _out_call(acc2, den2, r1(bg2), Wo1, r1(bo1), Wo2, r1(bo2))
    return y[:NND]


# speedup vs baseline: 21.1905x; 1.4570x over previous
import functools
import jax
import jax.numpy as jnp
from jax import lax
from jax.experimental import pallas as pl
from jax.experimental.pallas import tpu as pltpu
from jax.experimental.pallas import tpu_sc as plsc

NND = 10000
NPAD = 10240
NEDGE = 320000
EPAD = 331776
NW = 32
TPT = EPAD // NW
CHUNK = 48
SUPER = 8
NSUP = TPT // (SUPER * CHUNK)
ACCW = 128
STRIPE = NPAD // 16
BLK = 512
GRID = NPAD // BLK

_f32 = jnp.float32


def _lrelu(v, s=0.01):
    return jnp.maximum(v, v * s)


def _enc_body(des, tweet, num, cat, nf,
              Wd, bd, Wt, bt, Wn, bn, Wc, bc, Wf, bf, Wi, bi,
              W1, As1, Ad1,
              h0, h1o, h2o, h3o, alsT, aldT):
    d = _lrelu(jnp.dot(des[...], Wd[...], preferred_element_type=_f32) + bd[...])
    t = _lrelu(jnp.dot(tweet[...], Wt[...], preferred_element_type=_f32) + bt[...])
    n = _lrelu(jnp.dot(num[...], Wn[...], preferred_element_type=_f32) + bn[...])
    c = _lrelu(jnp.dot(cat[...], Wc[...], preferred_element_type=_f32) + bc[...])
    f = _lrelu(jnp.dot(nf[...], Wf[...], preferred_element_type=_f32) + bf[...])
    x = jnp.concatenate([d, t, n, c, f], axis=1)
    x = _lrelu(jnp.dot(x, Wi[...], preferred_element_type=_f32) + bi[...])
    h = jnp.dot(x, W1[...], preferred_element_type=_f32)
    h0[...] = h[:, 0:128]
    h1o[...] = h[:, 128:256]
    h2o[...] = h[:, 256:384]
    h3o[...] = h[:, 384:512]
    alsT[...] = lax.dot_general(As1[...], x, (((0,), (1,)), ((), ())),
                                preferred_element_type=_f32)
    aldT[...] = lax.dot_general(Ad1[...], x, (((0,), (1,)), ((), ())),
                                preferred_element_type=_f32)


def _full(shape):
    return pl.BlockSpec(shape, lambda i: (0,) * len(shape))


def _enc_call(des, tweet, num, cat, nf, Wd, bd, Wt, bt, Wn, bn, Wc, bc,
              Wf, bf, Wi, bi, W1, As1, Ad1):
    row = lambda w: pl.BlockSpec((BLK, w), lambda i: (i, 0))
    return pl.pallas_call(
        _enc_body,
        grid=(GRID,),
        in_specs=[row(768), row(768), row(7), row(11), row(1),
                  _full((768, 28)), _full((1, 28)),
                  _full((768, 36)), _full((1, 36)),
                  _full((7, 12)), _full((1, 12)),
                  _full((11, 40)), _full((1, 40)),
                  _full((1, 12)), _full((1, 12)),
                  _full((128, 128)), _full((1, 128)),
                  _full((128, 512)), _full((128, 4)), _full((128, 4))],
        out_specs=[row(128), row(128), row(128), row(128),
                   pl.BlockSpec((4, BLK), lambda i: (0, i)),
                   pl.BlockSpec((4, BLK), lambda i: (0, i))],
        out_shape=[jax.ShapeDtypeStruct((NPAD, 128), _f32)] * 4 +
                  [jax.ShapeDtypeStruct((4, NPAD), _f32)] * 2,
    )(des, tweet, num, cat, nf, Wd, bd, Wt, bt, Wn, bn, Wc, bc, Wf, bf,
      Wi, bi, W1, As1, Ad1)


def _make_gat_sc(heads):
    mesh = plsc.VectorSubcoreMesh(core_axis_name="c", subcore_axis_name="s")
    scratch = [
        pltpu.VMEM((NPAD,), _f32),
        pltpu.VMEM((NPAD,), _f32),
        pltpu.VMEM((SUPER * CHUNK,), jnp.int32),
        pltpu.VMEM((SUPER, CHUNK), jnp.int32),
        pltpu.VMEM((CHUNK, 128), _f32),
        pltpu.VMEM((CHUNK, 128), _f32),
        pltpu.VMEM((NPAD,), _f32),
        pltpu.VMEM_SHARED((NPAD, ACCW), _f32),
        pltpu.SemaphoreType.DMA,
        pltpu.SemaphoreType.DMA,
        pltpu.SemaphoreType.DMA,
        pltpu.SemaphoreType.DMA,
    ]

    def body(src_hbm, dst_hbm, alsT_hbm, aldT_hbm, t_hbm, zr_hbm, zd_hbm,
             out_hbm, den_hbm, als_v, ald_v, src_v, dst_v, buf0, buf1,
             den_v, acc_sh, gs0, gs1, ss0, ss1):
        bufs = (buf0, buf1)
        gsems = (gs0, gs1)
        ssems = (ss0, ss1)
        cid = lax.axis_index("c")
        sid = lax.axis_index("s")
        w = cid * 16 + sid
        ebase = w * TPT

        def head_pass(h, carry):
            pltpu.sync_copy(alsT_hbm.at[h], als_v)
            pltpu.sync_copy(aldT_hbm.at[h], ald_v)
            pltpu.sync_copy(zr_hbm, acc_sh.at[pl.ds(sid * STRIPE, STRIPE)])
            pltpu.sync_copy(zd_hbm, den_v)
            plsc.subcore_barrier()
            tab = t_hbm.at[h]

            def gather(j, par):
                off = pl.multiple_of(j * CHUNK, CHUNK)
                return pltpu.async_copy(
                    tab.at[src_v.at[pl.ds(off, CHUNK)]], bufs[par],
                    gsems[par])

            def process(j, par):
                buf = bufs[par]
                for g in range(3):
                    sidx = src_v[pl.ds(j * CHUNK + g * 16, 16)]
                    didx = dst_v[j, pl.ds(g * 16, 16)]
                    a = (plsc.load_gather(als_v, [sidx]) +
                         plsc.load_gather(ald_v, [didx]))
                    a = jnp.maximum(a, a * 0.2)
                    ex = jnp.exp(a)
                    lanes = lax.iota(jnp.int32, 16)
                    for e in range(16):
                        sc = ex[e]
                        r = g * 16 + e
                        for q in range(8):
                            sl = pl.ds(q * 16, 16)
                            buf[r, sl] = buf[r, sl] * sc
                        plsc.addupdate_scatter(den_v, [didx], ex,
                                               mask=lanes == e)
                return pltpu.async_copy(buf, acc_sh.at[dst_v.at[j]],
                                        ssems[par], add=True)

            def superchunk(si, c2):
                eoff = pl.multiple_of(ebase + si * (SUPER * CHUNK),
                                      SUPER * CHUNK)
                pltpu.sync_copy(src_hbm.at[pl.ds(eoff, SUPER * CHUNK)],
                                src_v)
                roff = pl.multiple_of((ebase // CHUNK) + si * SUPER, SUPER)
                pltpu.sync_copy(dst_hbm.at[pl.ds(roff, SUPER)], dst_v)

                def pair(pp, c3):
                    j0 = pp * 2
                    j1 = pp * 2 + 1
                    gd0 = gather(j0, 0)
                    gd1 = gather(j1, 1)
                    gd0.wait()
                    sd0 = process(j0, 0)
                    gd1.wait()
                    sd1 = process(j1, 1)
                    sd0.wait()
                    sd1.wait()
                    return c3

                lax.fori_loop(0, SUPER // 2, pair, 0)
                return c2

            lax.fori_loop(0, NSUP, superchunk, 0)
            plsc.subcore_barrier()
            pltpu.sync_copy(acc_sh.at[pl.ds(sid * STRIPE, STRIPE)],
                            out_hbm.at[h, cid, pl.ds(sid * STRIPE, STRIPE)])
            pltpu.sync_copy(den_v, den_hbm.at[h, w])
            plsc.subcore_barrier()
            return carry

        lax.fori_loop(0, heads, head_pass, 0)

    return pl.kernel(
        body,
        out_type=(jax.ShapeDtypeStruct((heads, 2, NPAD, ACCW), _f32),
                  jax.ShapeDtypeStruct((heads, NW, NPAD), _f32)),
        mesh=mesh,
        scratch_types=scratch,
        compiler_params=pltpu.CompilerParams(needs_layout_passes=False),
    )


def _mid_body(acc, dnn, bg1, W2, As2, Ad2, h2o, alsT, aldT):
    a = acc[...]
    d = dnn[...]
    xs = []
    for h in range(4):
        num = a[h, 0] + a[h, 1]
        den = jnp.sum(d[h], axis=0).reshape(BLK, 1)
        xs.append(num / (den + 1e-16))
    x1 = jnp.concatenate(xs, axis=1) + bg1[...]
    h2o[...] = jnp.dot(x1, W2[...], preferred_element_type=_f32)
    alsT[...] = lax.dot_general(As2[...], x1, (((0,), (1,)), ((), ())),
                                preferred_element_type=_f32)
    aldT[...] = lax.dot_general(Ad2[...], x1, (((0,), (1,)), ((), ())),
                                preferred_element_type=_f32)


def _mid_call(acc, dnn, bg1, W2, As2, Ad2):
    return pl.pallas_call(
        _mid_body,
        grid=(GRID,),
        in_specs=[pl.BlockSpec((4, 2, BLK, ACCW), lambda i: (0, 0, i, 0)),
                  pl.BlockSpec((4, NW, BLK), lambda i: (0, 0, i)),
                  _full((1, 512)), _full((512, 128)),
                  _full((512, 1)), _full((512, 1))],
        out_specs=[pl.BlockSpec((BLK, 128), lambda i: (i, 0)),
                   pl.BlockSpec((1, BLK), lambda i: (0, i)),
                   pl.BlockSpec((1, BLK), lambda i: (0, i))],
        out_shape=[jax.ShapeDtypeStruct((NPAD, 128), _f32),
                   jax.ShapeDtypeStruct((1, NPAD), _f32),
                   jax.ShapeDtypeStruct((1, NPAD), _f32)],
    )(acc, dnn, bg1, W2, As2, Ad2)


def _out_body(acc, dnn, bg2, Wo1, bo1, Wo2, bo2, y):
    a = acc[...]
    num = a[0, 0] + a[0, 1]
    den = jnp.sum(dnn[...][0], axis=0).reshape(BLK, 1)
    x2 = num / (den + 1e-16) + bg2[...]
    v = _lrelu(jnp.dot(x2, Wo1[...], preferred_element_type=_f32) + bo1[...])
    y[...] = jnp.dot(v, Wo2[...], preferred_element_type=_f32) + bo2[...]


def _out_call(acc, dnn, bg2, Wo1, bo1, Wo2, bo2):
    return pl.pallas_call(
        _out_body,
        grid=(GRID,),
        in_specs=[pl.BlockSpec((1, 2, BLK, ACCW), lambda i: (0, 0, i, 0)),
                  pl.BlockSpec((1, NW, BLK), lambda i: (0, 0, i)),
                  _full((1, 128)), _full((128, 128)), _full((1, 128)),
                  _full((128, 2)), _full((1, 2))],
        out_specs=pl.BlockSpec((BLK, 2), lambda i: (i, 0)),
        out_shape=jax.ShapeDtypeStruct((NPAD, 2), _f32),
    )(acc, dnn, bg2, Wo1, bo1, Wo2, bo2)


@jax.jit
def kernel(des, tweet, num_prop, cat_prop, new_feature, edge_index,
           Wd, bd, Wt, bt, Wn, bn, Wc, bc, Wf, bf, Wi, bi,
           W1, as1, ad1, bg1, W2, as2, ad2, bg2, Wo1, bo1, Wo2, bo2):
    f32 = _f32
    padn = lambda v: jnp.pad(v, ((0, NPAD - NND), (0, 0)))
    des_p = padn(des)
    tweet_p = padn(tweet)
    num_p = padn(num_prop)
    cat_p = padn(cat_prop)
    nf_p = padn(new_feature)

    loop = jnp.arange(NND, dtype=jnp.int32)
    src = jnp.concatenate([edge_index[0].astype(jnp.int32), loop,
                           jnp.zeros((EPAD - NEDGE - NND,), jnp.int32)])
    dst = jnp.concatenate([edge_index[1].astype(jnp.int32), loop,
                           jnp.full((EPAD - NEDGE - NND,), NND, jnp.int32)])
    dst = dst.reshape(EPAD // CHUNK, CHUNK)

    As1 = (W1.reshape(128, 4, 128) * as1).sum(-1)
    Ad1 = (W1.reshape(128, 4, 128) * ad1).sum(-1)
    As2 = (W2.reshape(512, 1, 128) * as2).sum(-1)
    Ad2 = (W2.reshape(512, 1, 128) * ad2).sum(-1)

    r1 = lambda b: b.reshape(1, -1).astype(f32)
    h0, h1, h2, h3, alsT1, aldT1 = _enc_call(
        des_p, tweet_p, num_p, cat_p, nf_p,
        Wd, r1(bd), Wt, r1(bt), Wn, r1(bn), Wc, r1(bc), Wf, r1(bf),
        Wi, r1(bi), W1, As1, Ad1)

    zr = jnp.zeros((STRIPE, ACCW), f32)
    zd = jnp.zeros((NPAD,), f32)
    t1 = jnp.stack([h0, h1, h2, h3])
    acc1, den1 = _make_gat_sc(4)(src, dst, alsT1, aldT1, t1, zr, zd)

    hx, alsT2, aldT2 = _mid_call(acc1, den1, r1(bg1), W2, As2, Ad2)

    acc2, den2 = _make_gat_sc(1)(src, dst, alsT2, aldT2, hx[None], zr, zd)

    y = _out_call(acc2, den2, r1(bg2), Wo1, r1(bo1), Wo2, r1(bo2))
    return y[:NND]
